# baseline (speedup 1.0000x reference)
import functools

import jax
import jax.numpy as jnp
from jax import lax
from jax.experimental import pallas as pl
from jax.experimental.pallas import tpu as pltpu

_KSIZE = 7
_PAD = 3


def _cbam_block_kernel(x_ref, fcw_ref, fcb_ref, sawc_ref, o_ref, *, H, W, NB):
    HW = H * W
    f32 = jnp.float32

    lane = lax.broadcasted_iota(jnp.int32, (1, HW), 1)
    if W & (W - 1) == 0:
        col = lane & (W - 1)
        row = lane >> (W.bit_length() - 1)
    else:
        col = lane % W
        row = lane // W

    pooled = [jnp.mean(x_ref[b], axis=1, keepdims=True, dtype=f32)
              for b in range(NB)]
    pooled_all = pooled[0] if NB == 1 else jnp.concatenate(pooled, axis=1)
    att = jnp.dot(fcw_ref[...], pooled_all,
                  preferred_element_type=f32) + fcb_ref[...]
    s_all = jax.nn.sigmoid(att)

    svecs = []
    means = []
    maxs = []
    for b in range(NB):
        sb = s_all if NB == 1 else s_all[:, b:b + 1]
        svecs.append(sb)
        y = x_ref[b] * sb
        means.append(jnp.mean(y, axis=0, keepdims=True))
        maxs.append(jnp.max(y, axis=0, keepdims=True))
    planes = jnp.concatenate(means + maxs, axis=0)

    qs = []
    for jj in range(_KSIZE):
        dj = jj - _PAD
        if dj == 0:
            qs.append(planes)
        else:
            q = pltpu.roll(planes, shift=(-dj) % HW, axis=1)
            qs.append(jnp.where((col >= -dj) & (col <= W - 1 - dj), q, 0.0))

    conv = jnp.zeros((2 * NB, HW), f32)
    for ii in range(_KSIZE):
        di = ii - _PAD
        t = jnp.zeros((2 * NB, HW), f32)
        for jj in range(_KSIZE):
            k = ii * _KSIZE + jj
            t = t + sawc_ref[:, k:k + 1] * qs[jj]
        if di == 0:
            conv = conv + t
        else:
            r = pltpu.roll(t, shift=(-di * W) % HW, axis=1)
            conv = conv + jnp.where((row >= -di) & (row <= H - 1 - di), r, 0.0)

    logits = conv[0:NB] + conv[NB:2 * NB]
    sa_all = jax.nn.sigmoid(logits)

    for b in range(NB):
        sab = sa_all[b:b + 1]
        o_ref[b] = ((x_ref[b] * sab) * svecs[b]).astype(o_ref.dtype)


def _pipelined_outer(x_hbm, fcw_ref, fcb_ref, sawc_ref, o_hbm,
                     *, H, W, NB, n_steps, in_bufs):
    HW = H * W
    C = fcw_ref.shape[0]

    def body(x_blk, o_blk):
        _cbam_block_kernel(x_blk, fcw_ref, fcb_ref, sawc_ref, o_blk,
                           H=H, W=W, NB=NB)

    pipe = pltpu.emit_pipeline(
        body,
        grid=(n_steps,),
        in_specs=[pl.BlockSpec(
            (NB, C, HW), lambda g: (g, 0, 0),
            pipeline_mode=pl.Buffered(buffer_count=in_bufs,
                                      use_lookahead=True))],
        out_specs=[pl.BlockSpec(
            (NB, C, HW), lambda g: (g, 0, 0),
            pipeline_mode=pl.Buffered(buffer_count=2))],
    )
    pipe(x_hbm, o_hbm)


def kernel(x, fc_w, fc_b, sa_w):
    B, C, H, W = x.shape
    HW = H * W
    NB = 4 if B % 4 == 0 else (2 if B % 2 == 0 else 1)
    IN_BUFS = 6

    x_flat = x.reshape(B, C, HW)
    fcw = fc_w.astype(jnp.float32)
    fcb = fc_b.reshape(C, 1).astype(jnp.float32)
    saw = sa_w.reshape(2, _KSIZE * _KSIZE).astype(jnp.float32)
    sawc = jnp.concatenate(
        [jnp.broadcast_to(saw[0:1], (NB, _KSIZE * _KSIZE)),
         jnp.broadcast_to(saw[1:2], (NB, _KSIZE * _KSIZE))], axis=0)

    blk_bytes = NB * C * HW * x.dtype.itemsize
    vmem_limit = int(min((IN_BUFS + 2 + 2) * blk_bytes + (12 << 20), 64 << 20))

    out_flat = pl.pallas_call(
        functools.partial(_pipelined_outer, H=H, W=W, NB=NB,
                          n_steps=B // NB, in_bufs=IN_BUFS),
        out_shape=jax.ShapeDtypeStruct((B, C, HW), x.dtype),
        in_specs=[
            pl.BlockSpec(memory_space=pl.ANY),
            pl.BlockSpec(memory_space=pltpu.MemorySpace.VMEM),
            pl.BlockSpec(memory_space=pltpu.MemorySpace.VMEM),
            pl.BlockSpec(memory_space=pltpu.MemorySpace.VMEM),
        ],
        out_specs=pl.BlockSpec(memory_space=pl.ANY),
        compiler_params=pltpu.CompilerParams(
            vmem_limit_bytes=vmem_limit),
    )(x_flat, fcw, fcb, sawc)
    return out_flat.reshape(B, C, H, W)
